# Initial kernel scaffold; baseline (speedup 1.0000x reference)
import functools

import jax
import jax.numpy as jnp
from jax import lax
from jax.experimental import pallas as pl
from jax.experimental.pallas import tpu as pltpu
from jax.experimental.pallas import tpu_sc as plsc

_NC = 2
_NS = 16
_NW = _NC * _NS

_CHUNK = 1024


@functools.lru_cache(maxsize=None)
def _make_lookup(n: int, v: int, d: int):
    assert n % (_NW * _CHUNK) == 0
    per_w = n // _NW
    n_chunks = per_w // _CHUNK
    mesh = plsc.VectorSubcoreMesh(core_axis_name="c", subcore_axis_name="s")

    @functools.partial(
        pl.kernel,
        mesh=mesh,
        out_type=jax.ShapeDtypeStruct((n, d), jnp.float32),
        scratch_types=[
            pltpu.VMEM((_CHUNK,), jnp.int32),
            pltpu.VMEM((_CHUNK, d), jnp.float32),
            pltpu.SemaphoreType.DMA,
        ],
    )
    def lookup(table_hbm, idx_hbm, out_hbm, idx_v, rows_v, sem):
        wid = lax.axis_index("s") * _NC + lax.axis_index("c")
        wbase = wid * per_w

        def body(c, carry):
            base = pl.multiple_of(wbase + c * _CHUNK, _CHUNK)
            pltpu.sync_copy(idx_hbm.at[pl.ds(base, _CHUNK)], idx_v)
            pltpu.async_copy(table_hbm.at[idx_v], rows_v, sem).wait()
            pltpu.sync_copy(rows_v, out_hbm.at[pl.ds(base, _CHUNK)])
            return carry

        lax.fori_loop(0, n_chunks, body, 0)

    return lookup


def kernel(x, weight):
    v, d = weight.shape
    idx = x.reshape(-1)
    out = _make_lookup(idx.shape[0], v, d)(weight, idx)
    return out.reshape(x.shape + (d,))


# speedup vs baseline: 1.4594x; 1.4594x over previous
import functools

import jax
import jax.numpy as jnp
from jax import lax
from jax.experimental import pallas as pl
from jax.experimental.pallas import tpu as pltpu
from jax.experimental.pallas import tpu_sc as plsc

_NC = 2
_NS = 16
_NW = _NC * _NS

_CHUNK = 1024


@functools.lru_cache(maxsize=None)
def _make_lookup(n: int, v: int, d: int):
    assert n % (_NW * _CHUNK) == 0
    per_w = n // _NW
    n_chunks = per_w // _CHUNK
    mesh = plsc.VectorSubcoreMesh(core_axis_name="c", subcore_axis_name="s")

    @functools.partial(
        pl.kernel,
        mesh=mesh,
        compiler_params=pltpu.CompilerParams(use_tc_tiling_on_sc=False),
        out_type=jax.ShapeDtypeStruct((n, d), jnp.float32),
        scratch_types=[
            pltpu.VMEM((_CHUNK,), jnp.int32),
            pltpu.VMEM((_CHUNK, d), jnp.float32),
            pltpu.SemaphoreType.DMA,
        ],
    )
    def lookup(table_hbm, idx_hbm, out_hbm, idx_v, rows_v, sem):
        wid = lax.axis_index("s") * _NC + lax.axis_index("c")
        wbase = wid * per_w

        def body(c, carry):
            base = pl.multiple_of(wbase + c * _CHUNK, _CHUNK)
            pltpu.sync_copy(idx_hbm.at[pl.ds(base, _CHUNK)], idx_v)
            pltpu.async_copy(table_hbm.at[idx_v], rows_v, sem).wait()
            pltpu.sync_copy(rows_v, out_hbm.at[pl.ds(base, _CHUNK)])
            return carry

        lax.fori_loop(0, n_chunks, body, 0)

    return lookup


def kernel(x, weight):
    v, d = weight.shape
    idx = x.reshape(-1)
    out = _make_lookup(idx.shape[0], v, d)(weight, idx)
    return out.reshape(x.shape + (d,))


# speedup vs baseline: 1.4923x; 1.0225x over previous
import functools

import jax
import jax.numpy as jnp
from jax import lax
from jax.experimental import pallas as pl
from jax.experimental.pallas import tpu as pltpu
from jax.experimental.pallas import tpu_sc as plsc

_NC = 2
_NS = 16
_NW = _NC * _NS

_CHUNK = 640
_NBUF = 4


@functools.lru_cache(maxsize=None)
def _make_lookup(n: int, v: int, d: int):
    assert n % (_NW * _CHUNK * _NBUF) == 0
    per_w = n // _NW
    n_chunks = per_w // _CHUNK
    n_groups = n_chunks // _NBUF
    mesh = plsc.VectorSubcoreMesh(core_axis_name="c", subcore_axis_name="s")

    @functools.partial(
        pl.kernel,
        mesh=mesh,
        compiler_params=pltpu.CompilerParams(use_tc_tiling_on_sc=False),
        out_type=jax.ShapeDtypeStruct((n, d), jnp.float32),
        scratch_types=[
            pltpu.VMEM((per_w,), jnp.int32),
            [pltpu.VMEM((_CHUNK, d), jnp.float32) for _ in range(_NBUF)],
            [pltpu.SemaphoreType.DMA for _ in range(_NBUF)],
            [pltpu.SemaphoreType.DMA for _ in range(_NBUF)],
        ],
    )
    def lookup(table_hbm, idx_hbm, out_hbm, idx_v, rows, gsem, ssem):
        wid = lax.axis_index("s") * _NC + lax.axis_index("c")
        wbase = wid * per_w

        pltpu.sync_copy(idx_hbm.at[pl.ds(wbase, per_w)], idx_v)

        def fire(c, b):
            off = pl.multiple_of(c * _CHUNK, _CHUNK)
            pltpu.async_copy(
                table_hbm.at[idx_v.at[pl.ds(off, _CHUNK)]], rows[b], gsem[b]
            )

        def drain(c, b):
            pltpu.make_async_copy(
                table_hbm.at[idx_v.at[pl.ds(0, _CHUNK)]], rows[b], gsem[b]
            ).wait()
            base = pl.multiple_of(wbase + c * _CHUNK, _CHUNK)
            pltpu.async_copy(rows[b], out_hbm.at[pl.ds(base, _CHUNK)], ssem[b])

        def wait_store(b):
            pltpu.make_async_copy(
                rows[b], out_hbm.at[pl.ds(wbase, _CHUNK)], ssem[b]
            ).wait()

        for b in range(_NBUF):
            fire(b, b)
        for b in range(_NBUF):
            drain(b, b)

        def group(g, carry):
            c0 = g * _NBUF
            for b in range(_NBUF):
                wait_store(b)
                fire(c0 + b, b)
            for b in range(_NBUF):
                drain(c0 + b, b)
            return carry

        lax.fori_loop(1, n_groups, group, 0)

        for b in range(_NBUF):
            wait_store(b)

    return lookup


def kernel(x, weight):
    v, d = weight.shape
    idx = x.reshape(-1)
    out = _make_lookup(idx.shape[0], v, d)(weight, idx)
    return out.reshape(x.shape + (d,))


# speedup vs baseline: 1.8587x; 1.2455x over previous
import functools

import jax
import jax.numpy as jnp
from jax import lax
from jax.experimental.layout import Layout, with_layout_constraint
from jax.experimental import pallas as pl
from jax.experimental.pallas import tpu as pltpu
from jax.experimental.pallas import tpu_sc as plsc

_NC = 2
_NS = 16
_NW = _NC * _NS

_BCHUNK = 4
_NBUF = 4


@functools.lru_cache(maxsize=None)
def _make_lookup(bsz: int, seq: int, v: int, d: int):
    assert bsz % (_NW * _BCHUNK * _NBUF) == 0
    chunk = _BCHUNK * seq
    b_per_w = bsz // _NW
    per_w = b_per_w * seq
    n_chunks = b_per_w // _BCHUNK
    n_groups = n_chunks // _NBUF
    mesh = plsc.VectorSubcoreMesh(core_axis_name="c", subcore_axis_name="s")

    @functools.partial(
        pl.kernel,
        mesh=mesh,
        compiler_params=pltpu.CompilerParams(use_tc_tiling_on_sc=False),
        out_type=jax.ShapeDtypeStruct((bsz, seq, d), jnp.float32),
        scratch_types=[
            pltpu.VMEM((per_w,), jnp.int32),
            [pltpu.VMEM((chunk, d), jnp.float32) for _ in range(_NBUF)],
            [pltpu.SemaphoreType.DMA for _ in range(_NBUF)],
            [pltpu.SemaphoreType.DMA for _ in range(_NBUF)],
        ],
    )
    def lookup(table2, idx_hbm, out3, idx_v, rows, gsem, ssem):
        wid = lax.axis_index("s") * _NC + lax.axis_index("c")
        wb = wid * b_per_w

        pltpu.sync_copy(idx_hbm.at[pl.ds(wb * seq, per_w)], idx_v)

        def fire(c, b):
            off = pl.multiple_of(c * chunk, chunk)
            pltpu.async_copy(
                table2.at[idx_v.at[pl.ds(off, chunk)]], rows[b], gsem[b]
            )

        def drain(c, b):
            pltpu.make_async_copy(
                table2.at[idx_v.at[pl.ds(0, chunk)]], rows[b], gsem[b]
            ).wait()
            brow = pl.multiple_of(wb + c * _BCHUNK, _BCHUNK)
            for k in range(_BCHUNK):
                pltpu.async_copy(
                    rows[b].at[pl.ds(k * seq, seq)], out3.at[brow + k], ssem[b]
                )

        def wait_store(b):
            pltpu.make_async_copy(
                table2.at[idx_v.at[pl.ds(0, chunk)]], rows[b], ssem[b]
            ).wait()

        for b in range(_NBUF):
            fire(b, b)
        for b in range(_NBUF):
            drain(b, b)

        def group(g, carry):
            c0 = g * _NBUF
            for b in range(_NBUF):
                wait_store(b)
                fire(c0 + b, b)
            for b in range(_NBUF):
                drain(c0 + b, b)
            return carry

        lax.fori_loop(1, n_groups, group, 0)

        for b in range(_NBUF):
            wait_store(b)

    return lookup


def kernel(x, weight):
    bsz, seq = x.shape
    v, d = weight.shape
    idx = x.reshape(-1)
    w_rm = with_layout_constraint(weight, Layout((0, 1)))
    return _make_lookup(bsz, seq, v, d)(w_rm, idx)
